# baseline (speedup 1.0000x reference)
import functools

import jax
import jax.numpy as jnp
from jax import lax
from jax.experimental import pallas as pl
from jax.experimental.pallas import tpu as pltpu
from jax.experimental.pallas import tpu_sc as plsc

VOCAB = 100000
EMB = 64
BATCH = 1024
CTX = 20

_NC = 2
_NS = 16
_NW = _NC * _NS
_BPW = BATCH // _NW
_IPW = _BPW * CTX
_GCH = 128
_NCH = _IPW // _GCH
_LANES = 16
_EVR = EMB // _LANES


def _pooled_sc(x_flat, emb_table):
    mesh = plsc.VectorSubcoreMesh(core_axis_name="c", subcore_axis_name="s")

    @functools.partial(
        pl.kernel,
        mesh=mesh,
        out_type=jax.ShapeDtypeStruct((BATCH, EMB), jnp.float32),
        scratch_types=[
            pltpu.VMEM((_IPW,), jnp.int32),
            pltpu.VMEM((_IPW, EMB), jnp.float32),
            pltpu.VMEM((_BPW, EMB), jnp.float32),
            pltpu.SemaphoreType.DMA,
        ],
        compiler_params=pltpu.CompilerParams(use_tc_tiling_on_sc=False),
    )
    def gather_pool(x_hbm, tab_hbm, pooled_hbm, idx_v, rows_v, out_v, sem):
        wid = lax.axis_index("s") * _NC + lax.axis_index("c")
        base = wid * _IPW
        pltpu.sync_copy(x_hbm.at[pl.ds(base, _IPW)], idx_v)
        cps = []
        for j in range(_NCH):
            cps.append(
                pltpu.async_copy(
                    tab_hbm.at[idx_v.at[pl.ds(j * _GCH, _GCH)]],
                    rows_v.at[pl.ds(j * _GCH, _GCH)],
                    sem,
                )
            )
        for cp in cps:
            cp.wait()

        def body(b, _):
            r0 = b * CTX
            for d in range(_EVR):
                acc = rows_v[r0, pl.ds(d * _LANES, _LANES)]
                for c in range(1, CTX):
                    acc = acc + rows_v[r0 + c, pl.ds(d * _LANES, _LANES)]
                out_v[b, pl.ds(d * _LANES, _LANES)] = acc
            return _

        lax.fori_loop(0, _BPW, body, None)
        pltpu.sync_copy(out_v, pooled_hbm.at[pl.ds(wid * _BPW, _BPW)])

    return gather_pool(x_flat, emb_table)


_VB = 2048


def _mm_body(p_ref, w_ref, o_ref):
    o_ref[...] = lax.dot_general(
        p_ref[...], w_ref[...],
        (((1,), (1,)), ((), ())),
        preferred_element_type=jnp.float32,
    )


def _project_tc(pooled, W):
    nb = pl.cdiv(VOCAB, _VB)
    return pl.pallas_call(
        _mm_body,
        grid=(nb,),
        in_specs=[
            pl.BlockSpec((BATCH, EMB), lambda i: (0, 0)),
            pl.BlockSpec((_VB, EMB), lambda i: (i, 0)),
        ],
        out_specs=pl.BlockSpec((BATCH, _VB), lambda i: (0, i)),
        out_shape=jax.ShapeDtypeStruct((BATCH, VOCAB), jnp.float32),
        compiler_params=pltpu.CompilerParams(
            dimension_semantics=("arbitrary",),
        ),
    )(pooled, W)


def kernel(x, emb_table, W):
    x_flat = x.astype(jnp.int32).reshape(-1)
    pooled = _pooled_sc(x_flat, emb_table)
    return _project_tc(pooled, W)


# speedup vs baseline: 2.7485x; 2.7485x over previous
import functools

import jax
import jax.numpy as jnp
from jax import lax
from jax.experimental import pallas as pl
from jax.experimental.pallas import tpu as pltpu
from jax.experimental.pallas import tpu_sc as plsc

VOCAB = 100000
EMB = 64
BATCH = 1024
CTX = 20

_NC = 2
_NS = 16
_NW = _NC * _NS
_BPW = BATCH // _NW
_IPW = _BPW * CTX
_GCH = 128
_NCH = _IPW // _GCH
_LANES = 16
_EVR = EMB // _LANES


def _pooled_sc(x_flat, emb_table):
    mesh = plsc.VectorSubcoreMesh(core_axis_name="c", subcore_axis_name="s")

    @functools.partial(
        pl.kernel,
        mesh=mesh,
        out_type=jax.ShapeDtypeStruct((BATCH, EMB), jnp.float32),
        scratch_types=[
            pltpu.VMEM((_IPW,), jnp.int32),
            pltpu.VMEM((_IPW, EMB), jnp.float32),
            pltpu.VMEM((_BPW, EMB), jnp.float32),
            pltpu.SemaphoreType.DMA,
        ],
        compiler_params=pltpu.CompilerParams(use_tc_tiling_on_sc=False),
    )
    def gather_pool(x_hbm, tab_hbm, pooled_hbm, idx_v, rows_v, out_v, sem):
        wid = lax.axis_index("s") * _NC + lax.axis_index("c")
        base = wid * _IPW
        pltpu.sync_copy(x_hbm.at[pl.ds(base, _IPW)], idx_v)
        cps = []
        for j in range(_NCH):
            cps.append(
                pltpu.async_copy(
                    tab_hbm.at[idx_v.at[pl.ds(j * _GCH, _GCH)]],
                    rows_v.at[pl.ds(j * _GCH, _GCH)],
                    sem,
                )
            )
        for cp in cps:
            cp.wait()

        def body(b, _):
            r0 = b * CTX
            for d in range(_EVR):
                acc = rows_v[r0, pl.ds(d * _LANES, _LANES)]
                for c in range(1, CTX):
                    acc = acc + rows_v[r0 + c, pl.ds(d * _LANES, _LANES)]
                out_v[b, pl.ds(d * _LANES, _LANES)] = acc
            return _

        lax.fori_loop(0, _BPW, body, None)
        pltpu.sync_copy(out_v, pooled_hbm.at[pl.ds(wid * _BPW, _BPW)])

    return gather_pool(x_flat, emb_table)


_VB = 2048


def _mm_body(wt_ref, p_ref, o_ref):
    o_ref[...] = lax.dot_general(
        wt_ref[...], p_ref[...],
        (((0,), (1,)), ((), ())),
        preferred_element_type=jnp.float32,
    )


def _project_tc(pooled, Wt):
    nb = pl.cdiv(VOCAB, _VB)
    return pl.pallas_call(
        _mm_body,
        grid=(nb,),
        in_specs=[
            pl.BlockSpec((EMB, _VB), lambda i: (0, i)),
            pl.BlockSpec((BATCH, EMB), lambda i: (0, 0)),
        ],
        out_specs=pl.BlockSpec((_VB, BATCH), lambda i: (i, 0)),
        out_shape=jax.ShapeDtypeStruct((VOCAB, BATCH), jnp.float32),
        compiler_params=pltpu.CompilerParams(
            dimension_semantics=("arbitrary",),
        ),
    )(Wt, pooled)


def kernel(x, emb_table, W):
    x_flat = x.astype(jnp.int32).reshape(-1)
    pooled = _pooled_sc(x_flat, emb_table)
    out_t = _project_tc(pooled, W.T)
    return out_t.T


# speedup vs baseline: 3.5659x; 1.2974x over previous
import functools

import jax
import jax.numpy as jnp
from jax import lax
from jax.experimental import pallas as pl
from jax.experimental.pallas import tpu as pltpu
from jax.experimental.pallas import tpu_sc as plsc

VOCAB = 100000
EMB = 64
BATCH = 1024
CTX = 20

_NC = 2
_NS = 16
_NW = _NC * _NS
_DPW = EMB // _NW
_LANES = 16
_NG = BATCH // _LANES


def _pooled_t_sc(xt, emb_t):
    mesh = plsc.VectorSubcoreMesh(core_axis_name="c", subcore_axis_name="s")

    @functools.partial(
        pl.kernel,
        mesh=mesh,
        out_type=jax.ShapeDtypeStruct((EMB, BATCH), jnp.float32),
        scratch_types=[
            pltpu.VMEM((CTX, BATCH), jnp.int32),
            pltpu.VMEM((VOCAB,), jnp.float32),
            pltpu.VMEM((_DPW, BATCH), jnp.float32),
        ],
        compiler_params=pltpu.CompilerParams(
            use_tc_tiling_on_sc=True, needs_layout_passes=False
        ),
    )
    def gather_pool(xt_hbm, tabt_hbm, pooled_hbm, xt_v, row_v, out_v):
        wid = lax.axis_index("s") * _NC + lax.axis_index("c")
        pltpu.sync_copy(xt_hbm, xt_v)
        for r in range(_DPW):
            d = wid * _DPW + r
            pltpu.sync_copy(tabt_hbm.at[d], row_v)

            def grp(g, _):
                b0 = g * _LANES
                acc = plsc.load_gather(row_v, [xt_v[0, pl.ds(b0, _LANES)]])
                for c in range(1, CTX):
                    acc = acc + plsc.load_gather(
                        row_v, [xt_v[c, pl.ds(b0, _LANES)]]
                    )
                out_v[r, pl.ds(b0, _LANES)] = acc
                return _

            lax.fori_loop(0, _NG, grp, None)
        pltpu.sync_copy(out_v, pooled_hbm.at[pl.ds(wid * _DPW, _DPW)])

    return gather_pool(xt, emb_t)


_VB = 2048


def _mm_body(wt_ref, pt_ref, o_ref):
    o_ref[...] = lax.dot_general(
        wt_ref[...], pt_ref[...],
        (((0,), (0,)), ((), ())),
        preferred_element_type=jnp.float32,
    )


def _project_tc(pooled_t, Wt):
    nb = pl.cdiv(VOCAB, _VB)
    return pl.pallas_call(
        _mm_body,
        grid=(nb,),
        in_specs=[
            pl.BlockSpec((EMB, _VB), lambda i: (0, i)),
            pl.BlockSpec((EMB, BATCH), lambda i: (0, 0)),
        ],
        out_specs=pl.BlockSpec((_VB, BATCH), lambda i: (i, 0)),
        out_shape=jax.ShapeDtypeStruct((VOCAB, BATCH), jnp.float32),
        compiler_params=pltpu.CompilerParams(
            dimension_semantics=("arbitrary",),
        ),
    )(Wt, pooled_t)


def kernel(x, emb_table, W):
    xt = x.astype(jnp.int32).T
    pooled_t = _pooled_t_sc(xt, emb_table.T)
    out_t = _project_tc(pooled_t, W.T)
    return out_t.T


# speedup vs baseline: 3.6058x; 1.0112x over previous
import functools

import jax
import jax.numpy as jnp
from jax import lax
from jax.experimental import pallas as pl
from jax.experimental.pallas import tpu as pltpu
from jax.experimental.pallas import tpu_sc as plsc

VOCAB = 100000
EMB = 64
BATCH = 1024
CTX = 20

_NC = 2
_NS = 16
_NW = _NC * _NS
_DPW = EMB // _NW
_LANES = 16
_NG = BATCH // _LANES


def _pooled_t_sc(xt, emb_t):
    mesh = plsc.VectorSubcoreMesh(core_axis_name="c", subcore_axis_name="s")

    @functools.partial(
        pl.kernel,
        mesh=mesh,
        out_type=jax.ShapeDtypeStruct((EMB, BATCH), jnp.float32),
        scratch_types=[
            pltpu.VMEM((CTX, BATCH), jnp.int32),
            pltpu.VMEM((VOCAB,), jnp.float32),
            pltpu.VMEM((_DPW, BATCH), jnp.float32),
        ],
        compiler_params=pltpu.CompilerParams(
            use_tc_tiling_on_sc=True, needs_layout_passes=False
        ),
    )
    def gather_pool(xt_hbm, tabt_hbm, pooled_hbm, xt_v, row_v, out_v):
        wid = lax.axis_index("s") * _NC + lax.axis_index("c")
        pltpu.sync_copy(xt_hbm, xt_v)
        for r in range(_DPW):
            d = wid * _DPW + r
            pltpu.sync_copy(tabt_hbm.at[d], row_v)

            def grp(g, _):
                b0 = g * _LANES
                acc = plsc.load_gather(row_v, [xt_v[0, pl.ds(b0, _LANES)]])
                for c in range(1, CTX):
                    acc = acc + plsc.load_gather(
                        row_v, [xt_v[c, pl.ds(b0, _LANES)]]
                    )
                out_v[r, pl.ds(b0, _LANES)] = acc
                return _

            lax.fori_loop(0, _NG, grp, None)
        pltpu.sync_copy(out_v, pooled_hbm.at[pl.ds(wid * _DPW, _DPW)])

    return gather_pool(xt, emb_t)


_VB = 4096


def _mm_body(wt_ref, pt_ref, o_ref):
    o_ref[...] = lax.dot_general(
        wt_ref[...], pt_ref[...],
        (((0,), (0,)), ((), ())),
        preferred_element_type=jnp.float32,
    )


def _project_tc(pooled_t, Wt):
    nb = pl.cdiv(VOCAB, _VB)
    return pl.pallas_call(
        _mm_body,
        grid=(nb,),
        in_specs=[
            pl.BlockSpec((EMB, _VB), lambda i: (0, i)),
            pl.BlockSpec((EMB, BATCH), lambda i: (0, 0)),
        ],
        out_specs=pl.BlockSpec((_VB, BATCH), lambda i: (i, 0)),
        out_shape=jax.ShapeDtypeStruct((VOCAB, BATCH), jnp.float32),
        compiler_params=pltpu.CompilerParams(
            dimension_semantics=("arbitrary",),
        ),
    )(Wt, pooled_t)


def kernel(x, emb_table, W):
    xt = x.astype(jnp.int32).T
    pooled_t = _pooled_t_sc(xt, emb_table.T)
    out_t = _project_tc(pooled_t, W.T)
    return out_t.T
